# baseline (speedup 1.0000x reference)
import jax
import jax.numpy as jnp
from jax.experimental import pallas as pl

N_EMB = 1024
EMB_DIM = 256
BETA = 0.25


def _vq_block_kernel(x_ref, emb_ref, e2_ref, out_ref, sse_ref):
    xb = x_ref[...]
    emb = emb_ref[...]
    x2 = jnp.sum(xb * xb, axis=1, keepdims=True)
    s = jax.lax.dot_general(
        xb, emb, (((1,), (1,)), ((), ())),
        preferred_element_type=jnp.float32)
    dist = (x2 + e2_ref[...]) - 2.0 * s
    mn = jnp.min(dist, axis=1, keepdims=True)
    iota = jax.lax.broadcasted_iota(jnp.int32, dist.shape, 1)
    idx = jnp.min(jnp.where(dist == mn, iota, N_EMB),
                  axis=1, keepdims=True)
    onehot = (iota == idx).astype(jnp.float32)
    ex = jax.lax.dot_general(
        onehot, emb, (((1,), (0,)), ((), ())),
        preferred_element_type=jnp.float32)
    out_ref[...] = ex

    @pl.when(pl.program_id(0) == 0)
    def _():
        sse_ref[...] = jnp.zeros_like(sse_ref)

    sse_ref[...] += jnp.sum(mn, axis=(0, 1), keepdims=True)


def kernel(x, embeddings):
    b, c, h, w = x.shape
    npix = b * h * w
    flat = jnp.transpose(x, (0, 2, 3, 1)).reshape(npix, c)
    e2 = jnp.sum(embeddings ** 2, axis=-1)[None, :]
    grid = 16
    bp = npix // grid
    out_flat, sse = pl.pallas_call(
        _vq_block_kernel,
        grid=(grid,),
        in_specs=[
            pl.BlockSpec((bp, c), lambda i: (i, 0)),
            pl.BlockSpec((N_EMB, c), lambda i: (0, 0)),
            pl.BlockSpec((1, N_EMB), lambda i: (0, 0)),
        ],
        out_specs=[
            pl.BlockSpec((bp, c), lambda i: (i, 0)),
            pl.BlockSpec((1, 1), lambda i: (0, 0)),
        ],
        out_shape=[
            jax.ShapeDtypeStruct((npix, c), jnp.float32),
            jax.ShapeDtypeStruct((1, 1), jnp.float32),
        ],
    )(flat, embeddings, e2)
    emb_out = jnp.transpose(out_flat.reshape(b, h, w, c), (0, 3, 1, 2))
    m = sse[0, 0] / (b * c * h * w)
    loss = m + BETA * m
    return (emb_out, loss)


# speedup vs baseline: 1.2081x; 1.2081x over previous
import jax
import jax.numpy as jnp
from jax.experimental import pallas as pl

N_EMB = 1024
EMB_DIM = 256
BETA = 0.25


def _vq_block_kernel(x_ref, emb_ref, e2_ref, out_ref, sse_ref):
    xb = x_ref[...]
    emb = emb_ref[...]
    x2 = jnp.sum(xb * xb, axis=1, keepdims=True)
    s = jax.lax.dot_general(
        xb, emb, (((1,), (1,)), ((), ())),
        preferred_element_type=jnp.float32)
    dist = (x2 + e2_ref[...]) - 2.0 * s
    mn = jnp.min(dist, axis=1, keepdims=True)
    iota = jax.lax.broadcasted_iota(jnp.int32, dist.shape, 1)
    idx = jnp.min(jnp.where(dist == mn, iota, N_EMB),
                  axis=1, keepdims=True)
    onehot = (iota == idx).astype(jnp.float32)
    ex = jax.lax.dot_general(
        onehot, emb, (((1,), (0,)), ((), ())),
        preferred_element_type=jnp.float32)
    out_ref[...] = ex

    @pl.when(pl.program_id(0) == 0)
    def _():
        sse_ref[...] = jnp.zeros_like(sse_ref)

    sse_ref[...] += jnp.sum(mn, axis=(0, 1), keepdims=True)


def kernel(x, embeddings):
    b, c, h, w = x.shape
    npix = b * h * w
    flat = jnp.transpose(x, (0, 2, 3, 1)).reshape(npix, c)
    e2 = jnp.sum(embeddings ** 2, axis=-1)[None, :]
    grid = 4
    bp = npix // grid
    out_flat, sse = pl.pallas_call(
        _vq_block_kernel,
        grid=(grid,),
        in_specs=[
            pl.BlockSpec((bp, c), lambda i: (i, 0)),
            pl.BlockSpec((N_EMB, c), lambda i: (0, 0)),
            pl.BlockSpec((1, N_EMB), lambda i: (0, 0)),
        ],
        out_specs=[
            pl.BlockSpec((bp, c), lambda i: (i, 0)),
            pl.BlockSpec((1, 1), lambda i: (0, 0)),
        ],
        out_shape=[
            jax.ShapeDtypeStruct((npix, c), jnp.float32),
            jax.ShapeDtypeStruct((1, 1), jnp.float32),
        ],
    )(flat, embeddings, e2)
    emb_out = jnp.transpose(out_flat.reshape(b, h, w, c), (0, 3, 1, 2))
    m = sse[0, 0] / (b * c * h * w)
    loss = m + BETA * m
    return (emb_out, loss)


# speedup vs baseline: 1.2261x; 1.0149x over previous
import jax
import jax.numpy as jnp
from jax.experimental import pallas as pl

N_EMB = 1024
EMB_DIM = 256
BETA = 0.25


def _vq_block_kernel(x_ref, emb_ref, e2_ref, out_ref, sse_ref):
    xb = x_ref[...]
    emb = emb_ref[...]
    x2 = jnp.sum(xb * xb, axis=1, keepdims=True)
    s = jax.lax.dot_general(
        xb, emb, (((1,), (1,)), ((), ())),
        preferred_element_type=jnp.float32)
    dist = (x2 + e2_ref[...]) - 2.0 * s
    mn = jnp.min(dist, axis=1, keepdims=True)
    iota = jax.lax.broadcasted_iota(jnp.int32, dist.shape, 1)
    idx = jnp.min(jnp.where(dist == mn, iota, N_EMB),
                  axis=1, keepdims=True)
    onehot = (iota == idx).astype(jnp.float32)
    ex = jax.lax.dot_general(
        onehot, emb, (((1,), (0,)), ((), ())),
        preferred_element_type=jnp.float32)
    out_ref[...] = ex

    @pl.when(pl.program_id(0) == 0)
    def _():
        sse_ref[...] = jnp.zeros_like(sse_ref)

    sse_ref[...] += jnp.sum(mn, axis=(0, 1), keepdims=True)


def kernel(x, embeddings):
    b, c, h, w = x.shape
    npix = b * h * w
    flat = jnp.transpose(x, (0, 2, 3, 1)).reshape(npix, c)
    e2 = jnp.sum(embeddings ** 2, axis=-1)[None, :]
    grid = 2
    bp = npix // grid
    out_flat, sse = pl.pallas_call(
        _vq_block_kernel,
        grid=(grid,),
        in_specs=[
            pl.BlockSpec((bp, c), lambda i: (i, 0)),
            pl.BlockSpec((N_EMB, c), lambda i: (0, 0)),
            pl.BlockSpec((1, N_EMB), lambda i: (0, 0)),
        ],
        out_specs=[
            pl.BlockSpec((bp, c), lambda i: (i, 0)),
            pl.BlockSpec((1, 1), lambda i: (0, 0)),
        ],
        out_shape=[
            jax.ShapeDtypeStruct((npix, c), jnp.float32),
            jax.ShapeDtypeStruct((1, 1), jnp.float32),
        ],
    )(flat, embeddings, e2)
    emb_out = jnp.transpose(out_flat.reshape(b, h, w, c), (0, 3, 1, 2))
    m = sse[0, 0] / (b * c * h * w)
    loss = m + BETA * m
    return (emb_out, loss)
